# Initial kernel scaffold; baseline (speedup 1.0000x reference)
import jax
import jax.numpy as jnp
from jax.experimental import pallas as pl


def kernel(x, edge_index, W, a_l, a_r, ln_gamma, ln_beta):
    raise NotImplementedError("write your pallas kernel here")


# speedup vs baseline: 5.3546x; 5.3546x over previous
import functools

import jax
import jax.numpy as jnp
from jax import lax
from jax.experimental import pallas as pl
from jax.experimental.pallas import tpu as pltpu
from jax.experimental.pallas import tpu_sc as plsc

_LANES = 16
_BB = 32


def _edge_hist_body(n, src_hbm, dst_hbm, zeros_hbm, out_hbm, src_v, dst_v, hist_v):
    wid = lax.axis_index("s") * 2 + lax.axis_index("c")
    num_edges = src_v.shape[0]

    @pl.when(wid == 0)
    def _():
        pltpu.sync_copy(src_hbm, src_v)
        pltpu.sync_copy(dst_hbm, dst_v)
        pltpu.sync_copy(zeros_hbm, hist_v)
        lane_base = lax.iota(jnp.int32, _LANES) * (n * n)
        ones = jnp.ones((_LANES,), jnp.float32)
        for i in range(num_edges // _LANES):
            s = src_v[pl.ds(i * _LANES, _LANES)]
            d = dst_v[pl.ds(i * _LANES, _LANES)]
            idx = lane_base + d * n + s
            cur = plsc.load_gather(hist_v, [idx])
            plsc.store_scatter(hist_v, [idx], cur + ones)
        pltpu.sync_copy(hist_v, out_hbm)


def _gat_tc_body(x_ref, w_ref, alr_ref, c16_ref, g_ref, b_ref, o_ref):
    Bb, N, d_in = x_ref.shape
    H = alr_ref.shape[1] // 2
    dh = d_in // H
    x3 = x_ref[...]
    x2 = x3.reshape(Bb * N, d_in)
    h2 = lax.dot_general(x2, w_ref[...], (((1,), (1,)), ((), ())),
                         preferred_element_type=jnp.float32)
    alpha = lax.dot_general(h2, alr_ref[...], (((1,), (0,)), ((), ())),
                            preferred_element_type=jnp.float32)
    al3 = alpha[:, :H].reshape(Bb, N, H)
    alT = jnp.swapaxes(al3, 1, 2)
    h3 = h2.reshape(Bb, N, d_in)
    C = jnp.sum(c16_ref[...], axis=0)
    Cb = C[None]
    mask = Cb > 0
    outs = []
    for h in range(H):
        al_h = alT[:, h:h + 1, :]
        ar_h = alpha[:, H + h:H + h + 1].reshape(Bb, N, 1)
        e = al_h + ar_h
        e = jnp.where(e > 0, e, 0.2 * e)
        em = jnp.where(mask, e, -1e30)
        m = jnp.max(em, axis=2, keepdims=True)
        esub = jnp.where(mask, e - m, -1e30)
        w = Cb * jnp.exp(esub)
        denom = jnp.sum(w, axis=2, keepdims=True)
        p = w / jnp.where(denom > 0, denom, 1.0)
        hh = h3[:, :, h * dh:(h + 1) * dh]
        outs.append(lax.dot_general(p, hh, (((2,), (1,)), ((0,), (0,))),
                                    preferred_element_type=jnp.float32))
    out = jnp.concatenate(outs, axis=-1) + x3
    mu = jnp.mean(out, -1, keepdims=True)
    var = jnp.mean((out - mu) ** 2, -1, keepdims=True)
    y = (out - mu) * lax.rsqrt(var + 1e-5) * g_ref[...] + b_ref[...]
    o_ref[...] = jnp.where(y > 0, y, jnp.exp(jnp.minimum(y, 0.0)) - 1.0)


def kernel(x, edge_index, W, a_l, a_r, ln_gamma, ln_beta):
    B, N, d_in = x.shape
    H, dh = a_l.shape
    d_out = H * dh
    E = edge_index.shape[1]
    src = edge_index[0]
    dst = edge_index[1]

    zeros = jnp.zeros((_LANES * N * N,), jnp.float32)
    mesh = plsc.VectorSubcoreMesh(core_axis_name="c", subcore_axis_name="s")
    c16 = pl.kernel(
        functools.partial(_edge_hist_body, N),
        out_type=jax.ShapeDtypeStruct((_LANES * N * N,), jnp.float32),
        mesh=mesh,
        scratch_types=[
            pltpu.VMEM((E,), jnp.int32),
            pltpu.VMEM((E,), jnp.int32),
            pltpu.VMEM((_LANES * N * N,), jnp.float32),
        ],
        compiler_params=pltpu.CompilerParams(needs_layout_passes=False),
    )(src, dst, zeros).reshape(_LANES, N, N)

    eye = jnp.eye(H, dtype=jnp.float32)
    Al = (eye[:, None, :] * a_l[:, :, None]).reshape(d_out, H)
    Ar = (eye[:, None, :] * a_r[:, :, None]).reshape(d_out, H)
    Alr = jnp.concatenate([Al, Ar], axis=1)
    g = ln_gamma.reshape(1, 1, d_out)
    b = ln_beta.reshape(1, 1, d_out)

    return pl.pallas_call(
        _gat_tc_body,
        grid=(B // _BB,),
        in_specs=[
            pl.BlockSpec((_BB, N, d_in), lambda i: (i, 0, 0)),
            pl.BlockSpec((d_out, d_in), lambda i: (0, 0)),
            pl.BlockSpec((d_in, 2 * H), lambda i: (0, 0)),
            pl.BlockSpec((_LANES, N, N), lambda i: (0, 0, 0)),
            pl.BlockSpec((1, 1, d_out), lambda i: (0, 0, 0)),
            pl.BlockSpec((1, 1, d_out), lambda i: (0, 0, 0)),
        ],
        out_specs=pl.BlockSpec((_BB, N, d_out), lambda i: (i, 0, 0)),
        out_shape=jax.ShapeDtypeStruct((B, N, d_out), jnp.float32),
        compiler_params=pltpu.CompilerParams(
            dimension_semantics=("arbitrary",)),
    )(x, W, Alr, c16, g, b)


# speedup vs baseline: 9.1425x; 1.7074x over previous
import functools

import jax
import jax.numpy as jnp
from jax import lax
from jax.experimental import pallas as pl
from jax.experimental.pallas import tpu as pltpu
from jax.experimental.pallas import tpu_sc as plsc

_LANES = 16
_BB = 64


def _edge_hist_body(n, src_hbm, dst_hbm, zeros_hbm, out_hbm, src_v, dst_v, hist_v):
    wid = lax.axis_index("s") * 2 + lax.axis_index("c")
    num_edges = src_v.shape[0]

    @pl.when(wid == 0)
    def _():
        pltpu.sync_copy(src_hbm, src_v)
        pltpu.sync_copy(dst_hbm, dst_v)
        pltpu.sync_copy(zeros_hbm, hist_v)
        lane_base = lax.iota(jnp.int32, _LANES) * (n * n)
        ones = jnp.ones((_LANES,), jnp.float32)
        for i in range(num_edges // _LANES):
            s = src_v[pl.ds(i * _LANES, _LANES)]
            d = dst_v[pl.ds(i * _LANES, _LANES)]
            idx = lane_base + d * n + s
            cur = plsc.load_gather(hist_v, [idx])
            plsc.store_scatter(hist_v, [idx], cur + ones)
        pltpu.sync_copy(hist_v, out_hbm)


def _gat_tc_body(x_ref, w_ref, alr_ref, rar_ref, rbd_ref, rbdt_ref, rout_ref,
                 c16_ref, g_ref, b_ref, o_ref):
    Bb, N, d_in = x_ref.shape
    H = alr_ref.shape[1] // 2
    dh = d_in // H
    GH = d_in // N
    x3 = x_ref[...]
    x2 = x3.reshape(Bb * N, d_in)
    h2 = lax.dot_general(x2, w_ref[...], (((1,), (1,)), ((), ())),
                         preferred_element_type=jnp.float32)
    alpha = lax.dot_general(h2, alr_ref[...], (((1,), (0,)), ((), ())),
                            preferred_element_type=jnp.float32)
    al3 = alpha[:, :H].reshape(Bb, N, H)
    alT = jnp.swapaxes(al3, 1, 2)
    mal = jnp.max(al3, axis=1)
    malN = jnp.broadcast_to(mal.reshape(Bb, 1, H), (Bb, N, H))
    bds = malN.reshape(Bb * N, H) + alpha[:, H:]
    bds = jnp.maximum(bds, 0.2 * bds)
    ar4 = lax.dot_general(alpha, rar_ref[...], (((1,), (0,)), ((), ())),
                          preferred_element_type=jnp.float32)
    ar4 = ar4.reshape(Bb, N, H * N)
    bd4 = lax.dot_general(bds, rbd_ref[...], (((1,), (0,)), ((), ())),
                          preferred_element_type=jnp.float32)
    bd4 = bd4.reshape(Bb, N, H * N)
    h3 = h2.reshape(Bb, N, d_in)
    C = jnp.sum(c16_ref[...], axis=0)
    Cfull = jnp.concatenate([C] * H, axis=-1)[None]
    outs = []
    w_groups = []
    for grp in range(H // GH):
        heads = range(grp * GH, (grp + 1) * GH)
        al4 = jnp.concatenate(
            [alT[:, h:h + 1, :].reshape(Bb, N) for h in heads], axis=-1
        ).reshape(Bb, 1, GH * N)
        sl = slice(grp * GH * N, (grp + 1) * GH * N)
        e = al4 + ar4[:, :, sl]
        e = jnp.maximum(e, 0.2 * e)
        w4 = Cfull[:, :, sl] * jnp.exp(e - bd4[:, :, sl])
        w_groups.append(w4)
        for j, h in enumerate(heads):
            w_h = w4[:, :, j * N:(j + 1) * N]
            hh = h3[:, :, h * dh:(h + 1) * dh]
            outs.append(lax.dot_general(w_h, hh, (((2,), (1,)), ((0,), (0,))),
                                        preferred_element_type=jnp.float32))
    w_all = jnp.concatenate(w_groups, axis=-1).reshape(Bb * N, H * N)
    den8 = lax.dot_general(w_all, rbdt_ref[...], (((1,), (0,)), ((), ())),
                           preferred_element_type=jnp.float32)
    dinv = 1.0 / jnp.where(den8 > 0, den8, 1.0)
    dfull = lax.dot_general(dinv, rout_ref[...], (((1,), (0,)), ((), ())),
                            preferred_element_type=jnp.float32)
    out = jnp.concatenate(outs, axis=-1) * dfull.reshape(Bb, N, d_in) + x3
    mu = jnp.mean(out, -1, keepdims=True)
    var = jnp.mean((out - mu) ** 2, -1, keepdims=True)
    y = (out - mu) * lax.rsqrt(var + 1e-5) * g_ref[...] + b_ref[...]
    o_ref[...] = jnp.where(y > 0, y, jnp.exp(jnp.minimum(y, 0.0)) - 1.0)


def kernel(x, edge_index, W, a_l, a_r, ln_gamma, ln_beta):
    B, N, d_in = x.shape
    H, dh = a_l.shape
    d_out = H * dh
    E = edge_index.shape[1]
    src = edge_index[0]
    dst = edge_index[1]

    zeros = jnp.zeros((_LANES * N * N,), jnp.float32)
    mesh = plsc.VectorSubcoreMesh(core_axis_name="c", subcore_axis_name="s")
    c16 = pl.kernel(
        functools.partial(_edge_hist_body, N),
        out_type=jax.ShapeDtypeStruct((_LANES * N * N,), jnp.float32),
        mesh=mesh,
        scratch_types=[
            pltpu.VMEM((E,), jnp.int32),
            pltpu.VMEM((E,), jnp.int32),
            pltpu.VMEM((_LANES * N * N,), jnp.float32),
        ],
        compiler_params=pltpu.CompilerParams(needs_layout_passes=False),
    )(src, dst, zeros).reshape(_LANES, N, N)

    eye = jnp.eye(H, dtype=jnp.float32)
    Al = (eye[:, None, :] * a_l[:, :, None]).reshape(d_out, H)
    Ar = (eye[:, None, :] * a_r[:, :, None]).reshape(d_out, H)
    Alr = jnp.concatenate([Al, Ar], axis=1)
    lane_head = jnp.arange(H * N) // N
    R_bd = (lane_head[None, :] == jnp.arange(H)[:, None]).astype(jnp.float32)
    R_ar = jnp.concatenate([jnp.zeros((H, H * N), jnp.float32), R_bd], axis=0)
    lane_out = jnp.arange(d_out) // dh
    R_out = (lane_out[None, :] == jnp.arange(H)[:, None]).astype(jnp.float32)
    g = ln_gamma.reshape(1, 1, d_out)
    b = ln_beta.reshape(1, 1, d_out)

    return pl.pallas_call(
        _gat_tc_body,
        grid=(B // _BB,),
        in_specs=[
            pl.BlockSpec((_BB, N, d_in), lambda i: (i, 0, 0)),
            pl.BlockSpec((d_out, d_in), lambda i: (0, 0)),
            pl.BlockSpec((d_in, 2 * H), lambda i: (0, 0)),
            pl.BlockSpec((2 * H, H * N), lambda i: (0, 0)),
            pl.BlockSpec((H, H * N), lambda i: (0, 0)),
            pl.BlockSpec((H * N, H), lambda i: (0, 0)),
            pl.BlockSpec((H, d_out), lambda i: (0, 0)),
            pl.BlockSpec((_LANES, N, N), lambda i: (0, 0, 0)),
            pl.BlockSpec((1, 1, d_out), lambda i: (0, 0, 0)),
            pl.BlockSpec((1, 1, d_out), lambda i: (0, 0, 0)),
        ],
        out_specs=pl.BlockSpec((_BB, N, d_out), lambda i: (i, 0, 0)),
        out_shape=jax.ShapeDtypeStruct((B, N, d_out), jnp.float32),
        compiler_params=pltpu.CompilerParams(
            dimension_semantics=("arbitrary",)),
    )(x, W, Alr, R_ar, R_bd, R_bd.T, R_out, c16, g, b)


# speedup vs baseline: 15.8158x; 1.7299x over previous
import functools

import jax
import jax.numpy as jnp
from jax import lax
from jax.experimental import pallas as pl
from jax.experimental.pallas import tpu as pltpu
from jax.experimental.pallas import tpu_sc as plsc

_LANES = 16
_BB = 64


def _edge_hist_body(n, src_hbm, dst_hbm, zeros_hbm, out_hbm, src_v, dst_v, hist_v):
    wid = lax.axis_index("s") * 2 + lax.axis_index("c")
    num_edges = src_v.shape[0]

    @pl.when(wid == 0)
    def _():
        pltpu.sync_copy(src_hbm, src_v)
        pltpu.sync_copy(dst_hbm, dst_v)
        pltpu.sync_copy(zeros_hbm, hist_v)
        lane_base = lax.iota(jnp.int32, _LANES) * (n * n)
        ones = jnp.ones((_LANES,), jnp.float32)
        for i in range(num_edges // _LANES):
            s = src_v[pl.ds(i * _LANES, _LANES)]
            d = dst_v[pl.ds(i * _LANES, _LANES)]
            idx = lane_base + d * n + s
            cur = plsc.load_gather(hist_v, [idx])
            plsc.store_scatter(hist_v, [idx], cur + ones)
        pltpu.sync_copy(hist_v, out_hbm)


def _gat_tc_body(x_ref, w_ref, alr_ref, rar_ref, rbd_ref, rbdt_ref, rout_ref,
                 c16_ref, g_ref, b_ref, o_ref):
    Bb, N, d_in = x_ref.shape
    H = alr_ref.shape[1] // 2
    dh = d_in // H
    GH = d_in // N
    x3 = x_ref[...]
    x2 = x3.reshape(Bb * N, d_in)
    h2 = lax.dot_general(x2, w_ref[...], (((1,), (1,)), ((), ())),
                         preferred_element_type=jnp.float32)
    alpha = lax.dot_general(h2, alr_ref[...], (((1,), (0,)), ((), ())),
                            preferred_element_type=jnp.float32)
    al3 = alpha[:, :H].reshape(Bb, N, H)
    alT = jnp.swapaxes(al3, 1, 2)
    mal = jnp.max(al3, axis=1)
    malN = jnp.broadcast_to(mal.reshape(Bb, 1, H), (Bb, N, H))
    bds = malN.reshape(Bb * N, H) + alpha[:, H:]
    bds = jnp.maximum(bds, 0.2 * bds)
    ar4 = lax.dot_general(alpha, rar_ref[...], (((1,), (0,)), ((), ())),
                          preferred_element_type=jnp.float32)
    ar4 = ar4.reshape(Bb, N, H * N)
    bd4 = lax.dot_general(bds, rbd_ref[...], (((1,), (0,)), ((), ())),
                          preferred_element_type=jnp.float32)
    bd4 = bd4.reshape(Bb, N, H * N)
    h3 = h2.reshape(Bb, N, d_in)
    C = jnp.sum(c16_ref[...], axis=0)
    Cfull = jnp.concatenate([C] * H, axis=-1)[None]
    outs = []
    w_groups = []
    for grp in range(H // GH):
        heads = range(grp * GH, (grp + 1) * GH)
        al4 = jnp.concatenate(
            [alT[:, h:h + 1, :].reshape(Bb, N) for h in heads], axis=-1
        ).reshape(Bb, 1, GH * N)
        sl = slice(grp * GH * N, (grp + 1) * GH * N)
        e = al4 + ar4[:, :, sl]
        e = jnp.maximum(e, 0.2 * e)
        w4 = Cfull[:, :, sl] * jnp.exp(e - bd4[:, :, sl])
        w_groups.append(w4)
        zpad = jnp.zeros((Bb, N, dh), jnp.float32)
        rows = []
        for j, h in enumerate(heads):
            hh = h3[:, :, h * dh:(h + 1) * dh]
            rows.append(jnp.concatenate(
                [zpad] * j + [hh] + [zpad] * (GH - 1 - j), axis=2))
        blk = jnp.concatenate(rows, axis=1)
        outs.append(lax.dot_general(w4, blk, (((2,), (1,)), ((0,), (0,))),
                                    preferred_element_type=jnp.float32))
    w_all = jnp.concatenate(w_groups, axis=-1).reshape(Bb * N, H * N)
    den8 = lax.dot_general(w_all, rbdt_ref[...], (((1,), (0,)), ((), ())),
                           preferred_element_type=jnp.float32)
    dinv = 1.0 / jnp.where(den8 > 0, den8, 1.0)
    dfull = lax.dot_general(dinv, rout_ref[...], (((1,), (0,)), ((), ())),
                            preferred_element_type=jnp.float32)
    out = jnp.concatenate(outs, axis=-1) * dfull.reshape(Bb, N, d_in) + x3
    mu = jnp.mean(out, -1, keepdims=True)
    var = jnp.mean((out - mu) ** 2, -1, keepdims=True)
    y = (out - mu) * lax.rsqrt(var + 1e-5) * g_ref[...] + b_ref[...]
    o_ref[...] = jnp.where(y > 0, y, jnp.exp(jnp.minimum(y, 0.0)) - 1.0)


def kernel(x, edge_index, W, a_l, a_r, ln_gamma, ln_beta):
    B, N, d_in = x.shape
    H, dh = a_l.shape
    d_out = H * dh
    E = edge_index.shape[1]
    src = edge_index[0]
    dst = edge_index[1]

    zeros = jnp.zeros((_LANES * N * N,), jnp.float32)
    mesh = plsc.VectorSubcoreMesh(core_axis_name="c", subcore_axis_name="s")
    c16 = pl.kernel(
        functools.partial(_edge_hist_body, N),
        out_type=jax.ShapeDtypeStruct((_LANES * N * N,), jnp.float32),
        mesh=mesh,
        scratch_types=[
            pltpu.VMEM((E,), jnp.int32),
            pltpu.VMEM((E,), jnp.int32),
            pltpu.VMEM((_LANES * N * N,), jnp.float32),
        ],
        compiler_params=pltpu.CompilerParams(needs_layout_passes=False),
    )(src, dst, zeros).reshape(_LANES, N, N)

    eye = jnp.eye(H, dtype=jnp.float32)
    Al = (eye[:, None, :] * a_l[:, :, None]).reshape(d_out, H)
    Ar = (eye[:, None, :] * a_r[:, :, None]).reshape(d_out, H)
    Alr = jnp.concatenate([Al, Ar], axis=1)
    lane_head = jnp.arange(H * N) // N
    R_bd = (lane_head[None, :] == jnp.arange(H)[:, None]).astype(jnp.float32)
    R_ar = jnp.concatenate([jnp.zeros((H, H * N), jnp.float32), R_bd], axis=0)
    lane_out = jnp.arange(d_out) // dh
    R_out = (lane_out[None, :] == jnp.arange(H)[:, None]).astype(jnp.float32)
    g = ln_gamma.reshape(1, 1, d_out)
    b = ln_beta.reshape(1, 1, d_out)

    return pl.pallas_call(
        _gat_tc_body,
        grid=(B // _BB,),
        in_specs=[
            pl.BlockSpec((_BB, N, d_in), lambda i: (i, 0, 0)),
            pl.BlockSpec((d_out, d_in), lambda i: (0, 0)),
            pl.BlockSpec((d_in, 2 * H), lambda i: (0, 0)),
            pl.BlockSpec((2 * H, H * N), lambda i: (0, 0)),
            pl.BlockSpec((H, H * N), lambda i: (0, 0)),
            pl.BlockSpec((H * N, H), lambda i: (0, 0)),
            pl.BlockSpec((H, d_out), lambda i: (0, 0)),
            pl.BlockSpec((_LANES, N, N), lambda i: (0, 0, 0)),
            pl.BlockSpec((1, 1, d_out), lambda i: (0, 0, 0)),
            pl.BlockSpec((1, 1, d_out), lambda i: (0, 0, 0)),
        ],
        out_specs=pl.BlockSpec((_BB, N, d_out), lambda i: (i, 0, 0)),
        out_shape=jax.ShapeDtypeStruct((B, N, d_out), jnp.float32),
        compiler_params=pltpu.CompilerParams(
            dimension_semantics=("arbitrary",)),
    )(x, W, Alr, R_ar, R_bd, R_bd.T, R_out, c16, g, b)


# speedup vs baseline: 16.6832x; 1.0548x over previous
import functools

import jax
import jax.numpy as jnp
from jax import lax
from jax.experimental import pallas as pl
from jax.experimental.pallas import tpu as pltpu
from jax.experimental.pallas import tpu_sc as plsc

_LANES = 16
_BB = 128


def _edge_hist_body(n, src_hbm, dst_hbm, zeros_hbm, out_hbm, src_v, dst_v, hist_v):
    wid = lax.axis_index("s") * 2 + lax.axis_index("c")
    num_edges = src_v.shape[0]

    @pl.when(wid == 0)
    def _():
        pltpu.sync_copy(src_hbm, src_v)
        pltpu.sync_copy(dst_hbm, dst_v)
        pltpu.sync_copy(zeros_hbm, hist_v)
        lane_base = lax.iota(jnp.int32, _LANES) * (n * n)
        ones = jnp.ones((_LANES,), jnp.float32)
        for i in range(num_edges // _LANES):
            s = src_v[pl.ds(i * _LANES, _LANES)]
            d = dst_v[pl.ds(i * _LANES, _LANES)]
            idx = lane_base + d * n + s
            cur = plsc.load_gather(hist_v, [idx])
            plsc.store_scatter(hist_v, [idx], cur + ones)
        pltpu.sync_copy(hist_v, out_hbm)


def _gat_tc_body(x_ref, w_ref, alr_ref, rar_ref, rbd_ref, rbdt_ref, rout_ref,
                 c16_ref, g_ref, b_ref, o_ref):
    Bb, N, d_in = x_ref.shape
    H = alr_ref.shape[1] // 2
    dh = d_in // H
    GH = d_in // N
    x3 = x_ref[...]
    x2 = x3.reshape(Bb * N, d_in)
    h2 = lax.dot_general(x2, w_ref[...], (((1,), (1,)), ((), ())),
                         preferred_element_type=jnp.float32)
    alpha = lax.dot_general(h2, alr_ref[...], (((1,), (0,)), ((), ())),
                            preferred_element_type=jnp.float32)
    al3 = alpha[:, :H].reshape(Bb, N, H)
    alT = jnp.swapaxes(al3, 1, 2)
    mal = jnp.max(al3, axis=1)
    malN = jnp.broadcast_to(mal.reshape(Bb, 1, H), (Bb, N, H))
    bds = malN.reshape(Bb * N, H) + alpha[:, H:]
    bds = jnp.maximum(bds, 0.2 * bds)
    ar4 = lax.dot_general(alpha, rar_ref[...], (((1,), (0,)), ((), ())),
                          preferred_element_type=jnp.float32)
    ar4 = ar4.reshape(Bb, N, H * N)
    bd4 = lax.dot_general(bds, rbd_ref[...], (((1,), (0,)), ((), ())),
                          preferred_element_type=jnp.float32)
    bd4 = bd4.reshape(Bb, N, H * N)
    h3 = h2.reshape(Bb, N, d_in)
    C = jnp.sum(c16_ref[...], axis=0)
    Cfull = jnp.concatenate([C] * H, axis=-1)[None]
    outs = []
    w_groups = []
    for grp in range(H // GH):
        heads = range(grp * GH, (grp + 1) * GH)
        al4 = jnp.concatenate(
            [alT[:, h:h + 1, :].reshape(Bb, N) for h in heads], axis=-1
        ).reshape(Bb, 1, GH * N)
        sl = slice(grp * GH * N, (grp + 1) * GH * N)
        e = al4 + ar4[:, :, sl]
        e = jnp.maximum(e, 0.2 * e)
        w4 = Cfull[:, :, sl] * jnp.exp(e - bd4[:, :, sl])
        w_groups.append(w4)
        zpad = jnp.zeros((Bb, N, dh), jnp.float32)
        rows = []
        for j, h in enumerate(heads):
            hh = h3[:, :, h * dh:(h + 1) * dh]
            rows.append(jnp.concatenate(
                [zpad] * j + [hh] + [zpad] * (GH - 1 - j), axis=2))
        blk = jnp.concatenate(rows, axis=1)
        outs.append(lax.dot_general(w4, blk, (((2,), (1,)), ((0,), (0,))),
                                    preferred_element_type=jnp.float32))
    w_all = jnp.concatenate(w_groups, axis=-1).reshape(Bb * N, H * N)
    den8 = lax.dot_general(w_all, rbdt_ref[...], (((1,), (0,)), ((), ())),
                           preferred_element_type=jnp.float32)
    dinv = 1.0 / jnp.where(den8 > 0, den8, 1.0)
    dfull = lax.dot_general(dinv, rout_ref[...], (((1,), (0,)), ((), ())),
                            preferred_element_type=jnp.float32)
    out = jnp.concatenate(outs, axis=-1) * dfull.reshape(Bb, N, d_in) + x3
    mu = jnp.mean(out, -1, keepdims=True)
    var = jnp.mean((out - mu) ** 2, -1, keepdims=True)
    y = (out - mu) * lax.rsqrt(var + 1e-5) * g_ref[...] + b_ref[...]
    o_ref[...] = jnp.where(y > 0, y, jnp.exp(jnp.minimum(y, 0.0)) - 1.0)


def kernel(x, edge_index, W, a_l, a_r, ln_gamma, ln_beta):
    B, N, d_in = x.shape
    H, dh = a_l.shape
    d_out = H * dh
    E = edge_index.shape[1]
    src = edge_index[0]
    dst = edge_index[1]

    zeros = jnp.zeros((_LANES * N * N,), jnp.float32)
    mesh = plsc.VectorSubcoreMesh(core_axis_name="c", subcore_axis_name="s")
    c16 = pl.kernel(
        functools.partial(_edge_hist_body, N),
        out_type=jax.ShapeDtypeStruct((_LANES * N * N,), jnp.float32),
        mesh=mesh,
        scratch_types=[
            pltpu.VMEM((E,), jnp.int32),
            pltpu.VMEM((E,), jnp.int32),
            pltpu.VMEM((_LANES * N * N,), jnp.float32),
        ],
        compiler_params=pltpu.CompilerParams(needs_layout_passes=False),
    )(src, dst, zeros).reshape(_LANES, N, N)

    eye = jnp.eye(H, dtype=jnp.float32)
    Al = (eye[:, None, :] * a_l[:, :, None]).reshape(d_out, H)
    Ar = (eye[:, None, :] * a_r[:, :, None]).reshape(d_out, H)
    Alr = jnp.concatenate([Al, Ar], axis=1)
    lane_head = jnp.arange(H * N) // N
    R_bd = (lane_head[None, :] == jnp.arange(H)[:, None]).astype(jnp.float32)
    R_ar = jnp.concatenate([jnp.zeros((H, H * N), jnp.float32), R_bd], axis=0)
    lane_out = jnp.arange(d_out) // dh
    R_out = (lane_out[None, :] == jnp.arange(H)[:, None]).astype(jnp.float32)
    g = ln_gamma.reshape(1, 1, d_out)
    b = ln_beta.reshape(1, 1, d_out)

    return pl.pallas_call(
        _gat_tc_body,
        grid=(B // _BB,),
        in_specs=[
            pl.BlockSpec((_BB, N, d_in), lambda i: (i, 0, 0)),
            pl.BlockSpec((d_out, d_in), lambda i: (0, 0)),
            pl.BlockSpec((d_in, 2 * H), lambda i: (0, 0)),
            pl.BlockSpec((2 * H, H * N), lambda i: (0, 0)),
            pl.BlockSpec((H, H * N), lambda i: (0, 0)),
            pl.BlockSpec((H * N, H), lambda i: (0, 0)),
            pl.BlockSpec((H, d_out), lambda i: (0, 0)),
            pl.BlockSpec((_LANES, N, N), lambda i: (0, 0, 0)),
            pl.BlockSpec((1, 1, d_out), lambda i: (0, 0, 0)),
            pl.BlockSpec((1, 1, d_out), lambda i: (0, 0, 0)),
        ],
        out_specs=pl.BlockSpec((_BB, N, d_out), lambda i: (i, 0, 0)),
        out_shape=jax.ShapeDtypeStruct((B, N, d_out), jnp.float32),
        compiler_params=pltpu.CompilerParams(
            dimension_semantics=("arbitrary",)),
    )(x, W, Alr, R_ar, R_bd, R_bd.T, R_out, c16, g, b)


# speedup vs baseline: 17.4522x; 1.0461x over previous
import functools

import jax
import jax.numpy as jnp
from jax import lax
from jax.experimental import pallas as pl
from jax.experimental.pallas import tpu as pltpu
from jax.experimental.pallas import tpu_sc as plsc

_LANES = 16
_BB = 256


def _edge_hist_body(n, src_hbm, dst_hbm, zeros_hbm, out_hbm, src_v, dst_v, hist_v):
    wid = lax.axis_index("s") * 2 + lax.axis_index("c")
    num_edges = src_v.shape[0]

    @pl.when(wid == 0)
    def _():
        pltpu.sync_copy(src_hbm, src_v)
        pltpu.sync_copy(dst_hbm, dst_v)
        pltpu.sync_copy(zeros_hbm, hist_v)
        lane_base = lax.iota(jnp.int32, _LANES) * (n * n)
        ones = jnp.ones((_LANES,), jnp.float32)
        for i in range(num_edges // _LANES):
            s = src_v[pl.ds(i * _LANES, _LANES)]
            d = dst_v[pl.ds(i * _LANES, _LANES)]
            idx = lane_base + d * n + s
            cur = plsc.load_gather(hist_v, [idx])
            plsc.store_scatter(hist_v, [idx], cur + ones)
        pltpu.sync_copy(hist_v, out_hbm)


def _gat_tc_body(x_ref, w_ref, alr_ref, rar_ref, rbd_ref, rbdt_ref, rout_ref,
                 c16_ref, g_ref, b_ref, o_ref):
    Bb, N, d_in = x_ref.shape
    H = alr_ref.shape[1] // 2
    dh = d_in // H
    GH = d_in // N
    x3 = x_ref[...]
    x2 = x3.reshape(Bb * N, d_in)
    h2 = lax.dot_general(x2, w_ref[...], (((1,), (1,)), ((), ())),
                         preferred_element_type=jnp.float32)
    alpha = lax.dot_general(h2, alr_ref[...], (((1,), (0,)), ((), ())),
                            preferred_element_type=jnp.float32)
    al3 = alpha[:, :H].reshape(Bb, N, H)
    alT = jnp.swapaxes(al3, 1, 2)
    mal = jnp.max(al3, axis=1)
    malN = jnp.broadcast_to(mal.reshape(Bb, 1, H), (Bb, N, H))
    bds = malN.reshape(Bb * N, H) + alpha[:, H:]
    bds = jnp.maximum(bds, 0.2 * bds)
    ar4 = lax.dot_general(alpha, rar_ref[...], (((1,), (0,)), ((), ())),
                          preferred_element_type=jnp.float32)
    ar4 = ar4.reshape(Bb, N, H * N)
    bd4 = lax.dot_general(bds, rbd_ref[...], (((1,), (0,)), ((), ())),
                          preferred_element_type=jnp.float32)
    bd4 = bd4.reshape(Bb, N, H * N)
    h3 = h2.reshape(Bb, N, d_in)
    C = jnp.sum(c16_ref[...], axis=0)
    Cfull = jnp.concatenate([C] * H, axis=-1)[None]
    outs = []
    w_groups = []
    for grp in range(H // GH):
        heads = range(grp * GH, (grp + 1) * GH)
        al4 = jnp.concatenate(
            [alT[:, h:h + 1, :].reshape(Bb, N) for h in heads], axis=-1
        ).reshape(Bb, 1, GH * N)
        sl = slice(grp * GH * N, (grp + 1) * GH * N)
        e = al4 + ar4[:, :, sl]
        e = jnp.maximum(e, 0.2 * e)
        w4 = Cfull[:, :, sl] * jnp.exp(e - bd4[:, :, sl])
        w_groups.append(w4)
        zpad = jnp.zeros((Bb, N, dh), jnp.float32)
        rows = []
        for j, h in enumerate(heads):
            hh = h3[:, :, h * dh:(h + 1) * dh]
            rows.append(jnp.concatenate(
                [zpad] * j + [hh] + [zpad] * (GH - 1 - j), axis=2))
        blk = jnp.concatenate(rows, axis=1)
        outs.append(lax.dot_general(w4, blk, (((2,), (1,)), ((0,), (0,))),
                                    preferred_element_type=jnp.float32))
    w_all = jnp.concatenate(w_groups, axis=-1).reshape(Bb * N, H * N)
    den8 = lax.dot_general(w_all, rbdt_ref[...], (((1,), (0,)), ((), ())),
                           preferred_element_type=jnp.float32)
    dinv = 1.0 / jnp.where(den8 > 0, den8, 1.0)
    dfull = lax.dot_general(dinv, rout_ref[...], (((1,), (0,)), ((), ())),
                            preferred_element_type=jnp.float32)
    out = jnp.concatenate(outs, axis=-1) * dfull.reshape(Bb, N, d_in) + x3
    mu = jnp.mean(out, -1, keepdims=True)
    var = jnp.mean((out - mu) ** 2, -1, keepdims=True)
    y = (out - mu) * lax.rsqrt(var + 1e-5) * g_ref[...] + b_ref[...]
    o_ref[...] = jnp.where(y > 0, y, jnp.exp(jnp.minimum(y, 0.0)) - 1.0)


def kernel(x, edge_index, W, a_l, a_r, ln_gamma, ln_beta):
    B, N, d_in = x.shape
    H, dh = a_l.shape
    d_out = H * dh
    E = edge_index.shape[1]
    src = edge_index[0]
    dst = edge_index[1]

    zeros = jnp.zeros((_LANES * N * N,), jnp.float32)
    mesh = plsc.VectorSubcoreMesh(core_axis_name="c", subcore_axis_name="s")
    c16 = pl.kernel(
        functools.partial(_edge_hist_body, N),
        out_type=jax.ShapeDtypeStruct((_LANES * N * N,), jnp.float32),
        mesh=mesh,
        scratch_types=[
            pltpu.VMEM((E,), jnp.int32),
            pltpu.VMEM((E,), jnp.int32),
            pltpu.VMEM((_LANES * N * N,), jnp.float32),
        ],
        compiler_params=pltpu.CompilerParams(needs_layout_passes=False),
    )(src, dst, zeros).reshape(_LANES, N, N)

    eye = jnp.eye(H, dtype=jnp.float32)
    Al = (eye[:, None, :] * a_l[:, :, None]).reshape(d_out, H)
    Ar = (eye[:, None, :] * a_r[:, :, None]).reshape(d_out, H)
    Alr = jnp.concatenate([Al, Ar], axis=1)
    lane_head = jnp.arange(H * N) // N
    R_bd = (lane_head[None, :] == jnp.arange(H)[:, None]).astype(jnp.float32)
    R_ar = jnp.concatenate([jnp.zeros((H, H * N), jnp.float32), R_bd], axis=0)
    lane_out = jnp.arange(d_out) // dh
    R_out = (lane_out[None, :] == jnp.arange(H)[:, None]).astype(jnp.float32)
    g = ln_gamma.reshape(1, 1, d_out)
    b = ln_beta.reshape(1, 1, d_out)

    return pl.pallas_call(
        _gat_tc_body,
        grid=(B // _BB,),
        in_specs=[
            pl.BlockSpec((_BB, N, d_in), lambda i: (i, 0, 0)),
            pl.BlockSpec((d_out, d_in), lambda i: (0, 0)),
            pl.BlockSpec((d_in, 2 * H), lambda i: (0, 0)),
            pl.BlockSpec((2 * H, H * N), lambda i: (0, 0)),
            pl.BlockSpec((H, H * N), lambda i: (0, 0)),
            pl.BlockSpec((H * N, H), lambda i: (0, 0)),
            pl.BlockSpec((H, d_out), lambda i: (0, 0)),
            pl.BlockSpec((_LANES, N, N), lambda i: (0, 0, 0)),
            pl.BlockSpec((1, 1, d_out), lambda i: (0, 0, 0)),
            pl.BlockSpec((1, 1, d_out), lambda i: (0, 0, 0)),
        ],
        out_specs=pl.BlockSpec((_BB, N, d_out), lambda i: (i, 0, 0)),
        out_shape=jax.ShapeDtypeStruct((B, N, d_out), jnp.float32),
        compiler_params=pltpu.CompilerParams(
            dimension_semantics=("arbitrary",)),
    )(x, W, Alr, R_ar, R_bd, R_bd.T, R_out, c16, g, b)
